# Initial kernel scaffold; baseline (speedup 1.0000x reference)
import jax
import jax.numpy as jnp
import numpy as np
from jax.experimental import pallas as pl
from jax.experimental.pallas import tpu as pltpu

B, S, D = 16, 128, 768
HEADS, E, TOPK = 12, 8, 2
DH = D // HEADS
_EPS = float(np.finfo(np.float64).eps)


def _cv_squared(x):
    eps = 1e-10
    n = x.shape[0]
    mean = jnp.sum(x) / n
    var = jnp.sum((x - mean) ** 2) / (n - 1)
    return var / (mean * mean + eps)


def _gating_kernel(q_ref, wg_ref, idx_ref, gates_ref, loss_ref):
    qc = q_ref[...]
    qc = jnp.where(jnp.isnan(qc), 0.0, qc)
    qs = jnp.sum(qc, axis=1)
    logits = jnp.dot(qs, wg_ref[...], preferred_element_type=jnp.float32)

    eye = jax.lax.broadcasted_iota(jnp.int32, (B, E), 1)
    m1 = jnp.max(logits, axis=1, keepdims=True)
    a1 = jnp.argmax(logits, axis=1)
    masked = jnp.where(eye == a1[:, None], -jnp.inf, logits)
    m2 = jnp.max(masked, axis=1, keepdims=True)
    a2 = jnp.argmax(masked, axis=1)
    z = jnp.exp(m2 - m1)
    g1 = 1.0 / (1.0 + z) + 1e-9
    g2 = z / (1.0 + z) + 1e-9

    gates_full = (jnp.where(eye == a1[:, None], g1, 0.0)
                  + jnp.where(eye == a2[:, None], g2, 0.0))
    importance = jnp.sum(gates_full, axis=0)
    load = jnp.sum((gates_full > 0.0).astype(jnp.float32), axis=0)
    loss_ref[0, 0] = (_cv_squared(importance) + _cv_squared(load)) * 0.01

    idx_ref[...] = jnp.stack([a1, a2], axis=1).astype(jnp.int32)
    gates_ref[...] = jnp.concatenate([g1, g2], axis=1)


def _mha_kernel(idx_ref, gates_ref, q_ref, k_ref, v_ref, mask_ref,
                wq_ref, wk_ref, wv_ref, wo_ref,
                bq_ref, bk_ref, bv_ref, bo_ref, out_ref):
    b = pl.program_id(0)
    s = pl.program_id(1)
    g = gates_ref[b, s]

    qb = q_ref[0]
    qb = jnp.where(jnp.isnan(qb), 0.0, qb)
    qp = jnp.dot(qb, wq_ref[0], preferred_element_type=jnp.float32) + bq_ref[0]
    kp = jnp.dot(k_ref[0], wk_ref[0], preferred_element_type=jnp.float32) + bk_ref[0]
    vp = jnp.dot(v_ref[0], wv_ref[0], preferred_element_type=jnp.float32) + bv_ref[0]

    qh = qp.reshape(S, HEADS, DH).transpose(1, 0, 2)
    kh = kp.reshape(S, HEADS, DH).transpose(1, 0, 2)
    vh = vp.reshape(S, HEADS, DH).transpose(1, 0, 2)

    scores = jax.lax.dot_general(
        qh, kh, (((2,), (2,)), ((0,), (0,))),
        preferred_element_type=jnp.float32) * (1.0 / np.sqrt(DH))
    scores = jnp.where(mask_ref[0, 0] == 0.0, -1e9, scores)
    scores = scores - jnp.max(scores, axis=-1, keepdims=True)
    ex = jnp.exp(scores)
    attn = ex / jnp.sum(ex, axis=-1, keepdims=True)

    ctx = jax.lax.dot_general(
        attn, vh, (((2,), (1,)), ((0,), (0,))),
        preferred_element_type=jnp.float32)
    ctx = ctx.transpose(1, 0, 2).reshape(S, D)
    out = jnp.dot(ctx, wo_ref[0], preferred_element_type=jnp.float32) + bo_ref[0]

    contrib = jnp.exp(out) * g

    @pl.when(s == 0)
    def _():
        out_ref[0] = contrib

    @pl.when(s == TOPK - 1)
    def _():
        tot = out_ref[0] + contrib
        tot = jnp.where(tot == 0.0, _EPS, tot)
        out_ref[0] = jnp.log(tot)


@jax.jit
def kernel(q, k, v, mask, w_gate, Wq, Wk, Wv, Wo, bq, bk, bv, bo):
    idx, gates2, loss = pl.pallas_call(
        _gating_kernel,
        out_shape=(
            jax.ShapeDtypeStruct((B, TOPK), jnp.int32),
            jax.ShapeDtypeStruct((B, TOPK), jnp.float32),
            jax.ShapeDtypeStruct((1, 1), jnp.float32),
        ),
    )(q, w_gate)

    def widx(b, s, idx_ref, gates_ref):
        return (idx_ref[b, s], 0, 0)

    def bidx(b, s, idx_ref, gates_ref):
        return (idx_ref[b, s], 0)

    combined = pl.pallas_call(
        _mha_kernel,
        grid_spec=pltpu.PrefetchScalarGridSpec(
            num_scalar_prefetch=2,
            grid=(B, TOPK),
            in_specs=[
                pl.BlockSpec((1, S, D), lambda b, s, i, g: (b, 0, 0)),
                pl.BlockSpec((1, S, D), lambda b, s, i, g: (b, 0, 0)),
                pl.BlockSpec((1, S, D), lambda b, s, i, g: (b, 0, 0)),
                pl.BlockSpec((1, 1, S, S), lambda b, s, i, g: (0, 0, 0, 0)),
                pl.BlockSpec((1, D, D), widx),
                pl.BlockSpec((1, D, D), widx),
                pl.BlockSpec((1, D, D), widx),
                pl.BlockSpec((1, D, D), widx),
                pl.BlockSpec((1, D), bidx),
                pl.BlockSpec((1, D), bidx),
                pl.BlockSpec((1, D), bidx),
                pl.BlockSpec((1, D), bidx),
            ],
            out_specs=pl.BlockSpec((1, S, D), lambda b, s, i, g: (b, 0, 0)),
        ),
        out_shape=jax.ShapeDtypeStruct((B, S, D), jnp.float32),
    )(idx, gates2, q, k, v, mask, Wq, Wk, Wv, Wo, bq, bk, bv, bo)

    return combined, loss[0, 0]


# speedup vs baseline: 2.4053x; 2.4053x over previous
import jax
import jax.numpy as jnp
import numpy as np
from jax.experimental import pallas as pl
from jax.experimental.pallas import tpu as pltpu

B, S, D = 16, 128, 768
HEADS, E, TOPK = 12, 8, 2
DH = D // HEADS
_EPS = float(np.finfo(np.float64).eps)


def _cv_squared(x):
    eps = 1e-10
    n = x.shape[0]
    mean = jnp.sum(x) / n
    var = jnp.sum((x - mean) ** 2) / (n - 1)
    return var / (mean * mean + eps)


def _gating_kernel(q_ref, wg_ref, idx_ref, gates_ref, loss_ref):
    qc = q_ref[...]
    qc = jnp.where(jnp.isnan(qc), 0.0, qc)
    qs = jnp.sum(qc, axis=1)
    logits = jnp.dot(qs, wg_ref[...], preferred_element_type=jnp.float32)

    eye = jax.lax.broadcasted_iota(jnp.int32, (B, E), 1)
    m1 = jnp.max(logits, axis=1, keepdims=True)
    a1 = jnp.argmax(logits, axis=1)
    masked = jnp.where(eye == a1[:, None], -jnp.inf, logits)
    m2 = jnp.max(masked, axis=1, keepdims=True)
    a2 = jnp.argmax(masked, axis=1)
    z = jnp.exp(m2 - m1)
    g1 = 1.0 / (1.0 + z) + 1e-9
    g2 = z / (1.0 + z) + 1e-9

    gates_full = (jnp.where(eye == a1[:, None], g1, 0.0)
                  + jnp.where(eye == a2[:, None], g2, 0.0))
    importance = jnp.sum(gates_full, axis=0)
    load = jnp.sum((gates_full > 0.0).astype(jnp.float32), axis=0)
    loss_val = (_cv_squared(importance) + _cv_squared(load)) * 0.01
    loss_ref[...] = jnp.reshape(loss_val, (1, 1))

    idx_ref[...] = jnp.stack([a1, a2], axis=1).astype(jnp.int32)
    gates_ref[...] = jnp.concatenate([g1, g2], axis=1)


def _mha_kernel(idx_ref, gates_ref, q_ref, k_ref, v_ref, mask_ref,
                wq_ref, wk_ref, wv_ref, wo_ref,
                bq_ref, bk_ref, bv_ref, bo_ref, out_ref):
    b = pl.program_id(0)
    s = pl.program_id(1)
    g = gates_ref[b, s]

    qb = q_ref[0]
    qb = jnp.where(jnp.isnan(qb), 0.0, qb)
    qp = jnp.dot(qb, wq_ref[0], preferred_element_type=jnp.float32) + bq_ref[0]
    kp = jnp.dot(k_ref[0], wk_ref[0], preferred_element_type=jnp.float32) + bk_ref[0]
    vp = jnp.dot(v_ref[0], wv_ref[0], preferred_element_type=jnp.float32) + bv_ref[0]

    qh = qp.reshape(S, HEADS, DH).transpose(1, 0, 2)
    kh = kp.reshape(S, HEADS, DH).transpose(1, 0, 2)
    vh = vp.reshape(S, HEADS, DH).transpose(1, 0, 2)

    scores = jax.lax.dot_general(
        qh, kh, (((2,), (2,)), ((0,), (0,))),
        preferred_element_type=jnp.float32) * (1.0 / np.sqrt(DH))
    scores = jnp.where(mask_ref[0, 0] == 0.0, -1e9, scores)
    scores = scores - jnp.max(scores, axis=-1, keepdims=True)
    ex = jnp.exp(scores)
    attn = ex / jnp.sum(ex, axis=-1, keepdims=True)

    ctx = jax.lax.dot_general(
        attn, vh, (((2,), (1,)), ((0,), (0,))),
        preferred_element_type=jnp.float32)
    ctx = ctx.transpose(1, 0, 2).reshape(S, D)
    out = jnp.dot(ctx, wo_ref[0], preferred_element_type=jnp.float32) + bo_ref[0]

    contrib = jnp.exp(out) * g

    @pl.when(s == 0)
    def _():
        out_ref[0] = contrib

    @pl.when(s == TOPK - 1)
    def _():
        tot = out_ref[0] + contrib
        tot = jnp.where(tot == 0.0, _EPS, tot)
        out_ref[0] = jnp.log(tot)


@jax.jit
def kernel(q, k, v, mask, w_gate, Wq, Wk, Wv, Wo, bq, bk, bv, bo):
    idx, gates2, loss = pl.pallas_call(
        _gating_kernel,
        out_shape=(
            jax.ShapeDtypeStruct((B, TOPK), jnp.int32),
            jax.ShapeDtypeStruct((B, TOPK), jnp.float32),
            jax.ShapeDtypeStruct((1, 1), jnp.float32),
        ),
    )(q, w_gate)

    def widx(b, s, idx_ref, gates_ref):
        return (idx_ref[b, s], 0, 0)

    def bidx(b, s, idx_ref, gates_ref):
        return (idx_ref[b, s], 0, 0)

    bq3 = bq.reshape(E, 1, D)
    bk3 = bk.reshape(E, 1, D)
    bv3 = bv.reshape(E, 1, D)
    bo3 = bo.reshape(E, 1, D)

    combined = pl.pallas_call(
        _mha_kernel,
        grid_spec=pltpu.PrefetchScalarGridSpec(
            num_scalar_prefetch=2,
            grid=(B, TOPK),
            in_specs=[
                pl.BlockSpec((1, S, D), lambda b, s, i, g: (b, 0, 0)),
                pl.BlockSpec((1, S, D), lambda b, s, i, g: (b, 0, 0)),
                pl.BlockSpec((1, S, D), lambda b, s, i, g: (b, 0, 0)),
                pl.BlockSpec((1, 1, S, S), lambda b, s, i, g: (0, 0, 0, 0)),
                pl.BlockSpec((1, D, D), widx),
                pl.BlockSpec((1, D, D), widx),
                pl.BlockSpec((1, D, D), widx),
                pl.BlockSpec((1, D, D), widx),
                pl.BlockSpec((1, 1, D), bidx),
                pl.BlockSpec((1, 1, D), bidx),
                pl.BlockSpec((1, 1, D), bidx),
                pl.BlockSpec((1, 1, D), bidx),
            ],
            out_specs=pl.BlockSpec((1, S, D), lambda b, s, i, g: (b, 0, 0)),
        ),
        out_shape=jax.ShapeDtypeStruct((B, S, D), jnp.float32),
    )(idx, gates2, q, k, v, mask, Wq, Wk, Wv, Wo, bq3, bk3, bv3, bo3)

    return combined, loss[0, 0]
